# Initial kernel scaffold; baseline (speedup 1.0000x reference)
import jax
import jax.numpy as jnp
from jax.experimental import pallas as pl


def kernel(x, table, W_up):
    raise NotImplementedError("write your pallas kernel here")


# speedup vs baseline: 15.5558x; 15.5558x over previous
import functools

import jax
import jax.numpy as jnp
from jax import lax
from jax.experimental import pallas as pl
from jax.experimental.pallas import tpu as pltpu
from jax.experimental.pallas import tpu_sc as plsc


def _pack_body(tt_ref, out_ref):
    t = tt_ref[...]
    bw = t.shape[1]
    out_ref[...] = t.T.reshape(bw // 4, 4 * t.shape[0])


def _tc_pack_table(tableT, BW=2048):
    D, V = tableT.shape
    grid = (pl.cdiv(V, BW),)
    return pl.pallas_call(
        _pack_body,
        grid=grid,
        in_specs=[pl.BlockSpec((D, BW), lambda g: (0, g))],
        out_specs=pl.BlockSpec((BW // 4, 4 * D), lambda g: (g, 0)),
        out_shape=jax.ShapeDtypeStruct((V // 4, 4 * D), jnp.float32),
    )(tableT)


def _sc_gather(N, V, D):
    NW = 32
    n_w = N // NW
    C = 1024
    n_chunks = n_w // C
    mesh = plsc.VectorSubcoreMesh(core_axis_name="c", subcore_axis_name="s")

    @functools.partial(
        pl.kernel,
        mesh=mesh,
        compiler_params=pltpu.CompilerParams(use_tc_tiling_on_sc=False),
        out_type=jax.ShapeDtypeStruct((N, D), jnp.float32),
        scratch_types=[
            pltpu.VMEM((C,), jnp.int32),
            pltpu.VMEM((C, D), jnp.float32),
            pltpu.SemaphoreType.DMA,
        ],
    )
    def gather_kernel(idx_hbm, table_hbm, out_hbm, idx_v, rows_v, sem):
        wid = lax.axis_index("s") * 2 + lax.axis_index("c")
        base = wid * n_w

        def body(i, carry):
            off = base + i * C
            pltpu.sync_copy(idx_hbm.at[pl.ds(off, C)], idx_v)
            pltpu.async_copy(table_hbm.at[idx_v], rows_v, sem).wait()
            pltpu.sync_copy(rows_v, out_hbm.at[pl.ds(off, C)])
            return carry

        lax.fori_loop(0, n_chunks, body, 0)

    return gather_kernel


def _mm_body(emb_ref, w4_ref, out_ref):
    out_ref[...] = jnp.dot(
        emb_ref[...], w4_ref[...], preferred_element_type=jnp.float32
    )


def _tc_matmul(emb128, W4, BT4=512):
    M, K = emb128.shape
    _, Nw = W4.shape
    grid = (M // BT4,)
    return pl.pallas_call(
        _mm_body,
        grid=grid,
        in_specs=[
            pl.BlockSpec((BT4, K), lambda g: (g, 0)),
            pl.BlockSpec((K, Nw), lambda g: (0, 0)),
        ],
        out_specs=pl.BlockSpec((BT4, Nw), lambda g: (g, 0)),
        out_shape=jax.ShapeDtypeStruct((M, Nw), jnp.float32),
    )(emb128, W4)


def kernel(x, table, W_up):
    B, L = x.shape
    V, D = table.shape
    E = W_up.shape[0]
    N = B * L

    idxT = jnp.transpose(x).reshape(N)

    embT = _sc_gather(N, V, D)(idxT, table)

    emb128 = embT.reshape(N // 4, 4 * D)
    W4 = jnp.kron(jnp.eye(4, dtype=W_up.dtype), W_up.T)
    out512 = _tc_matmul(emb128, W4)

    return out512.reshape(L, B, E).transpose(1, 0, 2)


# speedup vs baseline: 25.9768x; 1.6699x over previous
import functools

import jax
import jax.numpy as jnp
from jax import lax
from jax.experimental import pallas as pl
from jax.experimental.pallas import tpu as pltpu
from jax.experimental.pallas import tpu_sc as plsc


def _proj_body(tt_ref, w_ref, out_ref):
    out_ref[...] = lax.dot_general(
        tt_ref[...],
        w_ref[...],
        (((0,), (0,)), ((), ())),
        preferred_element_type=jnp.float32,
    )


def _tc_project(tableT, WT, BW=2048):
    D, V = tableT.shape
    E = WT.shape[1]
    grid = (pl.cdiv(V, BW),)
    return pl.pallas_call(
        _proj_body,
        grid=grid,
        in_specs=[
            pl.BlockSpec((D, BW), lambda g: (0, g)),
            pl.BlockSpec((D, E), lambda g: (0, 0)),
        ],
        out_specs=pl.BlockSpec((BW, E), lambda g: (g, 0)),
        out_shape=jax.ShapeDtypeStruct((V, E), jnp.float32),
    )(tableT, WT)


def _sc_gather(N, V, E):
    NW = 32
    n_w = N // NW
    C = 512
    n_chunks = n_w // C
    mesh = plsc.VectorSubcoreMesh(core_axis_name="c", subcore_axis_name="s")

    @functools.partial(
        pl.kernel,
        mesh=mesh,
        compiler_params=pltpu.CompilerParams(use_tc_tiling_on_sc=False),
        out_type=jax.ShapeDtypeStruct((N, E), jnp.float32),
        scratch_types=[
            pltpu.VMEM((C,), jnp.int32),
            pltpu.VMEM((C, E), jnp.float32),
            pltpu.SemaphoreType.DMA,
        ],
    )
    def gather_kernel(idx_hbm, ptab_hbm, out_hbm, idx_v, rows_v, sem):
        wid = lax.axis_index("s") * 2 + lax.axis_index("c")
        base = wid * n_w

        def body(i, carry):
            off = base + i * C
            pltpu.sync_copy(idx_hbm.at[pl.ds(off, C)], idx_v)
            pltpu.async_copy(ptab_hbm.at[idx_v], rows_v, sem).wait()
            pltpu.sync_copy(rows_v, out_hbm.at[pl.ds(off, C)])
            return carry

        lax.fori_loop(0, n_chunks, body, 0)

    return gather_kernel


def kernel(x, table, W_up):
    B, L = x.shape
    V, D = table.shape
    E = W_up.shape[0]
    N = B * L

    idxT = jnp.transpose(x).reshape(N)

    tableT = jnp.transpose(table)
    WT = jnp.transpose(W_up)
    ptab = _tc_project(tableT, WT)

    outT = _sc_gather(N, V, E)(idxT, ptab)

    return outT.reshape(L, B, E).transpose(1, 0, 2)


# speedup vs baseline: 27.4370x; 1.0562x over previous
import functools

import jax
import jax.numpy as jnp
from jax import lax
from jax.experimental import pallas as pl
from jax.experimental.pallas import tpu as pltpu
from jax.experimental.pallas import tpu_sc as plsc


def _proj_body(tt_ref, w_ref, out_ref):
    out_ref[...] = lax.dot_general(
        tt_ref[...],
        w_ref[...],
        (((0,), (0,)), ((), ())),
        preferred_element_type=jnp.float32,
    )


def _tc_project(tableT, WT, BW=2048):
    D, V = tableT.shape
    E = WT.shape[1]
    grid = (pl.cdiv(V, BW),)
    return pl.pallas_call(
        _proj_body,
        grid=grid,
        in_specs=[
            pl.BlockSpec((D, BW), lambda g: (0, g)),
            pl.BlockSpec((D, E), lambda g: (0, 0)),
        ],
        out_specs=pl.BlockSpec((BW, E), lambda g: (g, 0)),
        out_shape=jax.ShapeDtypeStruct((V, E), jnp.float32),
    )(tableT, WT)


def _sc_gather(N, V, E):
    NW = 32
    n_w = N // NW
    C = 400
    n_chunks = n_w // C
    assert n_chunks >= 2
    mesh = plsc.VectorSubcoreMesh(core_axis_name="c", subcore_axis_name="s")

    @functools.partial(
        pl.kernel,
        mesh=mesh,
        compiler_params=pltpu.CompilerParams(use_tc_tiling_on_sc=False),
        out_type=jax.ShapeDtypeStruct((N, E), jnp.float32),
        scratch_types=[
            pltpu.VMEM((2, C), jnp.int32),
            pltpu.VMEM((2, C, E), jnp.float32),
            pltpu.SemaphoreType.DMA,
            pltpu.SemaphoreType.DMA,
        ],
    )
    def gather_kernel(idx_hbm, ptab_hbm, out_hbm, idx_v, rows_v, gsem, wsem):
        wid = lax.axis_index("s") * 2 + lax.axis_index("c")
        base = wid * n_w

        def load_and_gather(i):
            b = i % 2
            pltpu.sync_copy(idx_hbm.at[pl.ds(base + i * C, C)], idx_v.at[b])
            pltpu.async_copy(ptab_hbm.at[idx_v.at[b]], rows_v.at[b], gsem)

        def gather_wait(i):
            b = i % 2
            pltpu.make_async_copy(
                ptab_hbm.at[idx_v.at[b]], rows_v.at[b], gsem
            ).wait()

        def write_start(i):
            b = i % 2
            pltpu.async_copy(
                rows_v.at[b], out_hbm.at[pl.ds(base + i * C, C)], wsem
            )

        def write_wait(i):
            b = i % 2
            pltpu.make_async_copy(
                rows_v.at[b], out_hbm.at[pl.ds(base + i * C, C)], wsem
            ).wait()

        load_and_gather(0)
        load_and_gather(1)

        def body(i, carry):
            gather_wait(i)
            write_start(i)

            @pl.when(i + 2 < n_chunks)
            def _():
                write_wait(i)
                load_and_gather(i + 2)

            return carry

        lax.fori_loop(0, n_chunks, body, 0)
        write_wait(n_chunks - 2)
        write_wait(n_chunks - 1)

    return gather_kernel


def kernel(x, table, W_up):
    B, L = x.shape
    V, D = table.shape
    E = W_up.shape[0]
    N = B * L

    idxT = jnp.transpose(x).reshape(N)

    tableT = jnp.transpose(table)
    WT = jnp.transpose(W_up)
    ptab = _tc_project(tableT, WT)

    outT = _sc_gather(N, V, E)(idxT, ptab)

    return outT.reshape(L, B, E).transpose(1, 0, 2)


# speedup vs baseline: 33.5126x; 1.2214x over previous
import functools

import jax
import jax.numpy as jnp
from jax import lax
from jax.experimental import pallas as pl
from jax.experimental.pallas import tpu as pltpu
from jax.experimental.pallas import tpu_sc as plsc


def _proj_body(tt_ref, w_ref, out_ref):
    out_ref[...] = lax.dot_general(
        tt_ref[...],
        w_ref[...],
        (((0,), (0,)), ((), ())),
        preferred_element_type=jnp.float32,
    )


def _tc_project(tableT, WT, BW=4096):
    D, V = tableT.shape
    E = WT.shape[1]
    grid = (pl.cdiv(V, BW),)
    return pl.pallas_call(
        _proj_body,
        grid=grid,
        in_specs=[
            pl.BlockSpec((D, BW), lambda g: (0, g)),
            pl.BlockSpec((D, E), lambda g: (0, 0)),
        ],
        out_specs=pl.BlockSpec((BW, E), lambda g: (g, 0)),
        out_shape=jax.ShapeDtypeStruct((V, E), jnp.float32),
        compiler_params=pltpu.CompilerParams(
            fuse_transposed_lhs_in_matmul=True
        ),
    )(tableT, WT)


def _sc_gather(N, V, E):
    NW = 32
    n_w = N // NW
    C = 400
    n_chunks = n_w // C
    assert n_chunks >= 2
    mesh = plsc.VectorSubcoreMesh(core_axis_name="c", subcore_axis_name="s")

    @functools.partial(
        pl.kernel,
        mesh=mesh,
        compiler_params=pltpu.CompilerParams(use_tc_tiling_on_sc=False),
        out_type=jax.ShapeDtypeStruct((N, E), jnp.float32),
        scratch_types=[
            pltpu.VMEM((2, C), jnp.int32),
            pltpu.VMEM((2, C, E), jnp.float32),
            pltpu.SemaphoreType.DMA,
            pltpu.SemaphoreType.DMA,
        ],
    )
    def gather_kernel(idx_hbm, ptab_hbm, out_hbm, idx_v, rows_v, gsem, wsem):
        wid = lax.axis_index("s") * 2 + lax.axis_index("c")
        base = wid * n_w

        def load_and_gather(i):
            b = i % 2
            pltpu.sync_copy(idx_hbm.at[pl.ds(base + i * C, C)], idx_v.at[b])
            pltpu.async_copy(ptab_hbm.at[idx_v.at[b]], rows_v.at[b], gsem)

        def gather_wait(i):
            b = i % 2
            pltpu.make_async_copy(
                ptab_hbm.at[idx_v.at[b]], rows_v.at[b], gsem
            ).wait()

        def write_start(i):
            b = i % 2
            pltpu.async_copy(
                rows_v.at[b], out_hbm.at[pl.ds(base + i * C, C)], wsem
            )

        def write_wait(i):
            b = i % 2
            pltpu.make_async_copy(
                rows_v.at[b], out_hbm.at[pl.ds(base + i * C, C)], wsem
            ).wait()

        load_and_gather(0)
        load_and_gather(1)

        def body(i, carry):
            gather_wait(i)
            write_start(i)

            @pl.when(i + 2 < n_chunks)
            def _():
                write_wait(i)
                load_and_gather(i + 2)

            return carry

        lax.fori_loop(0, n_chunks, body, 0)
        write_wait(n_chunks - 2)
        write_wait(n_chunks - 1)

    return gather_kernel


def kernel(x, table, W_up):
    B, L = x.shape
    V, D = table.shape
    E = W_up.shape[0]
    N = B * L

    idxT = jnp.transpose(x).reshape(N)

    tableT = jnp.transpose(table)
    WT = jnp.transpose(W_up)
    ptab = _tc_project(tableT, WT)

    outT = _sc_gather(N, V, E)(idxT, ptab)

    return outT.reshape(L, B, E).transpose(1, 0, 2)
